# baseline (speedup 1.0000x reference)
import functools

import jax
import jax.numpy as jnp
import numpy as np
from jax import lax
from jax.experimental import pallas as pl
from jax.experimental.pallas import tpu as pltpu
from jax.experimental.pallas import tpu_sc as plsc

_ATOMS = 51
_V_MAX = 10.0
_V_MIN = -10.0
_SKEW = 0.0

_NTILES = 32
_SLICE_STARTS = (0, 16, 32, 35)

_LOG2_POLY = (
    -0.02456854, 0.2650243, -1.229291, 3.2128003,
    -5.2616825, 6.0668716, -3.0291514,
)
_LN2 = 0.6931471805599453


def _col_coeffs():
    atoms = _ATOMS
    delta = np.float32((_V_MAX - _V_MIN) / (atoms - 1))
    supports = np.linspace(_V_MIN, _V_MAX, atoms).astype(np.float32)
    tz = np.clip(np.float32(_SKEW) + supports, _V_MIN, _V_MAX).astype(np.float32)
    b = ((tz - np.float32(_V_MIN)) / delta).astype(np.float32)
    l = np.floor(b)
    u = np.ceil(b)
    l = np.where((u > 0) & (l == u), l - 1.0, l).astype(np.float32)
    u = np.where((l < atoms - 1) & (l == u), u + 1.0, u).astype(np.float32)
    wl = (u - b).astype(np.float32)
    wu = (b - l).astype(np.float32)
    j = np.arange(atoms, dtype=np.float32)
    l_is_j = l == j
    cs = np.where(l_is_j, wl, wu).astype(np.float32)
    cl = np.where(l_is_j, 0.0, wl).astype(np.float32)
    cr = np.where(l_is_j, wu, 0.0).astype(np.float32)
    return cs, cl, cr


def _coeff_table():
    cs, cl, cr = _col_coeffs()
    tab = np.zeros((12, 16), dtype=np.float32)
    for s, c0 in enumerate(_SLICE_STARTS):
        cols = np.arange(c0, c0 + 16)
        keep = (cols < _ATOMS) if s < 3 else (cols >= 48)
        for g, arr in enumerate((cs, cl, cr)):
            vals = np.where(keep & (cols < _ATOMS), arr[np.minimum(cols, _ATOMS - 1)], 0.0)
            tab[4 * g + s] = vals.astype(np.float32)
    return tab


def _log16(x):
    xi = lax.bitcast_convert_type(x, jnp.int32)
    e = ((xi >> 23) & 0xFF) - 127
    m = lax.bitcast_convert_type((xi & 0x007FFFFF) | 0x3F800000, jnp.float32)
    acc = jnp.full((16,), _LOG2_POLY[0], jnp.float32)
    for c in _LOG2_POLY[1:]:
        acc = acc * m + jnp.float32(c)
    return (e.astype(jnp.float32) + acc) * jnp.float32(_LN2)


_CHUNK_ROWS = 256


_UNROLL = 4


def _sc_body(rows_per_tile, a_hbm, f_hbm, coef_hbm, out_hbm,
             a_v, f_v, coef_v, halo, accv):
    nc = 2
    wid = lax.axis_index("s") * nc + lax.axis_index("c")
    base = wid * rows_per_tile
    pltpu.sync_copy(coef_hbm, coef_v)

    zeros = jnp.zeros((16,), jnp.float32)
    for u in range(_UNROLL):
        halo[u, pl.ds(0, 16)] = zeros
        halo[u, pl.ds(48, 16)] = zeros
    cs = [coef_v[s] for s in range(4)]
    cl = [coef_v[4 + s] for s in range(4)]
    cr = [coef_v[8 + s] for s in range(4)]

    def group_body(g_idx, accs):
        r0 = g_idx * _UNROLL
        for u in range(_UNROLL):
            for s, c0 in enumerate(_SLICE_STARTS):
                x = f_v[r0 + u, pl.ds(c0, 16)] + jnp.float32(1e-16)
                halo[u, pl.ds(1 + c0, 16)] = _log16(x)
        out = []
        for u in range(_UNROLL):
            acc = accs[u]
            for s, c0 in enumerate(_SLICE_STARTS):
                a = a_v[r0 + u, pl.ds(c0, 16)]
                lm = halo[u, pl.ds(c0, 16)]
                lc = halo[u, pl.ds(c0 + 1, 16)]
                lp = halo[u, pl.ds(c0 + 2, 16)]
                acc = acc + a * (cs[s] * lc + cl[s] * lm + cr[s] * lp)
            out.append(acc)
        return tuple(out)

    chunk = min(_CHUNK_ROWS, rows_per_tile)
    accs = (zeros,) * _UNROLL
    for ci in range(rows_per_tile // chunk):
        rows = pl.ds(base + ci * chunk, chunk)
        pltpu.sync_copy(a_hbm.at[rows], a_v)
        pltpu.sync_copy(f_hbm.at[rows], f_v)
        accs = lax.fori_loop(0, chunk // _UNROLL, group_body, accs)
    acc = (accs[0] + accs[1]) + (accs[2] + accs[3])
    accv[...] = acc
    pltpu.sync_copy(accv, out_hbm.at[wid])


def _sc_loss_partials(anchor, feature, nrows):
    atoms = anchor.shape[1]
    rows_per_tile = nrows // _NTILES
    coef = jnp.asarray(_coeff_table())
    mesh = plsc.VectorSubcoreMesh(core_axis_name="c", subcore_axis_name="s")
    run = pl.kernel(
        functools.partial(_sc_body, rows_per_tile),
        out_type=jax.ShapeDtypeStruct((_NTILES, 16), jnp.float32),
        mesh=mesh,
        scratch_types=[
            pltpu.VMEM((min(_CHUNK_ROWS, rows_per_tile), atoms), jnp.float32),
            pltpu.VMEM((min(_CHUNK_ROWS, rows_per_tile), atoms), jnp.float32),
            pltpu.VMEM((12, 16), jnp.float32),
            pltpu.VMEM((_UNROLL, 64), jnp.float32),
            pltpu.VMEM((16,), jnp.float32),
        ],
    )
    return run(anchor, feature, coef)


def _proj_matrix():
    cs, cl, cr = _col_coeffs()
    w = np.zeros((_ATOMS, _ATOMS), dtype=np.float32)
    for j in range(_ATOMS):
        w[j, j] += cs[j]
        if j > 0:
            w[j, j - 1] += cl[j]
        if j < _ATOMS - 1:
            w[j, j + 1] += cr[j]
    return w


def _tc_body(anchor_ref, feature_ref, w_ref, out_ref):
    i = pl.program_id(0)
    logf = jnp.log(feature_ref[...] + 1e-16)
    proj = lax.dot_general(
        anchor_ref[...], w_ref[...],
        dimension_numbers=(((1,), (0,)), ((), ())),
        preferred_element_type=jnp.float32,
    )
    partial = jnp.sum(proj * logf, keepdims=True)

    @pl.when(i == 0)
    def _init():
        out_ref[...] = jnp.zeros_like(out_ref)

    out_ref[...] += partial


def _tc_loss_sum(anchor, feature, row0, nrows, bm):
    batch, atoms = anchor.shape
    w = jnp.asarray(_proj_matrix())
    blk0 = row0 // bm
    total = pl.pallas_call(
        _tc_body,
        grid=(nrows // bm,),
        in_specs=[
            pl.BlockSpec((bm, atoms), lambda i: (i + blk0, 0)),
            pl.BlockSpec((bm, atoms), lambda i: (i + blk0, 0)),
            pl.BlockSpec((atoms, atoms), lambda i: (0, 0)),
        ],
        out_specs=pl.BlockSpec((1, 1), lambda i: (0, 0)),
        out_shape=jax.ShapeDtypeStruct((1, 1), jnp.float32),
    )(anchor, feature, w)
    return total[0, 0]


_SC_ROWS = 16384


def kernel(anchor, feature):
    batch, _ = anchor.shape
    sc_partials = _sc_loss_partials(anchor, feature, _SC_ROWS)
    if batch > _SC_ROWS:
        tc_total = _tc_loss_sum(anchor, feature, _SC_ROWS, batch - _SC_ROWS, 2048)
    else:
        tc_total = jnp.float32(0.0)
    total = tc_total + jnp.sum(sc_partials)
    return (-total / batch).astype(jnp.float32)


# speedup vs baseline: 1.5995x; 1.5995x over previous
import functools

import jax
import jax.numpy as jnp
import numpy as np
from jax import lax
from jax.experimental import pallas as pl
from jax.experimental.pallas import tpu as pltpu
from jax.experimental.pallas import tpu_sc as plsc

_ATOMS = 51
_V_MAX = 10.0
_V_MIN = -10.0
_SKEW = 0.0

_NTILES = 32
_SLICE_STARTS = (0, 16, 32, 35)

_LOG2_POLY = (
    -0.02456854, 0.2650243, -1.229291, 3.2128003,
    -5.2616825, 6.0668716, -3.0291514,
)
_LN2 = 0.6931471805599453


def _col_coeffs():
    atoms = _ATOMS
    delta = np.float32((_V_MAX - _V_MIN) / (atoms - 1))
    supports = np.linspace(_V_MIN, _V_MAX, atoms).astype(np.float32)
    tz = np.clip(np.float32(_SKEW) + supports, _V_MIN, _V_MAX).astype(np.float32)
    b = ((tz - np.float32(_V_MIN)) / delta).astype(np.float32)
    l = np.floor(b)
    u = np.ceil(b)
    l = np.where((u > 0) & (l == u), l - 1.0, l).astype(np.float32)
    u = np.where((l < atoms - 1) & (l == u), u + 1.0, u).astype(np.float32)
    wl = (u - b).astype(np.float32)
    wu = (b - l).astype(np.float32)
    j = np.arange(atoms, dtype=np.float32)
    l_is_j = l == j
    cs = np.where(l_is_j, wl, wu).astype(np.float32)
    cl = np.where(l_is_j, 0.0, wl).astype(np.float32)
    cr = np.where(l_is_j, wu, 0.0).astype(np.float32)
    return cs, cl, cr


def _coeff_table():
    cs, cl, cr = _col_coeffs()
    atoms = _ATOMS
    csk = cs.copy()
    clp = np.concatenate([cl[1:], [np.float32(0.0)]]).astype(np.float32)
    crm = np.concatenate([[np.float32(0.0)], cr[:-1]]).astype(np.float32)
    tab = np.zeros((12, 16), dtype=np.float32)
    for s, c0 in enumerate(_SLICE_STARTS):
        cols = np.arange(c0, c0 + 16)
        keep = (cols < atoms) if s < 3 else (cols >= 48)
        for g, arr in enumerate((csk, clp, crm)):
            vals = np.where(keep & (cols < atoms), arr[np.minimum(cols, atoms - 1)], 0.0)
            tab[4 * g + s] = vals.astype(np.float32)
    return tab


def _log16(x):
    xi = lax.bitcast_convert_type(x, jnp.int32)
    e = ((xi >> 23) & 0xFF) - 127
    m = lax.bitcast_convert_type((xi & 0x007FFFFF) | 0x3F800000, jnp.float32)
    acc = jnp.full((16,), _LOG2_POLY[0], jnp.float32)
    for c in _LOG2_POLY[1:]:
        acc = acc * m + jnp.float32(c)
    return (e.astype(jnp.float32) + acc) * jnp.float32(_LN2)


_CHUNK_ROWS = 256


_UNROLL = 4


def _gather16(v, idx):
    return lax.gather(
        v, idx.reshape(16, 1),
        dimension_numbers=lax.GatherDimensionNumbers(
            offset_dims=(), collapsed_slice_dims=(0,), start_index_map=(0,)),
        slice_sizes=(1,),
        mode=lax.GatherScatterMode.PROMISE_IN_BOUNDS,
    )


def _sc_body(rows_per_tile, a_hbm, f_hbm, coef_hbm, out_hbm,
             a_v, f_v, coef_v, accv):
    nc = 2
    wid = lax.axis_index("s") * nc + lax.axis_index("c")
    base = wid * rows_per_tile
    pltpu.sync_copy(coef_hbm, coef_v)

    zeros = jnp.zeros((16,), jnp.float32)
    csk = [coef_v[s] for s in range(4)]
    clp = [coef_v[4 + s] for s in range(4)]
    crm = [coef_v[8 + s] for s in range(4)]
    lanes = lax.iota(jnp.int32, 16)
    idx_m = jnp.maximum(lanes - 1, 0)
    idx_p = jnp.minimum(lanes + 1, 15)

    def row_terms(r, acc):
        for s, c0 in enumerate(_SLICE_STARTS):
            logf = _log16(f_v[r, pl.ds(c0, 16)] + jnp.float32(1e-16))
            a0 = a_v[r, pl.ds(c0, 16)]
            if s == 0:
                am = _gather16(a0, idx_m)
            else:
                am = a_v[r, pl.ds(c0 - 1, 16)]
            if s == 3:
                ap = _gather16(a0, idx_p)
            else:
                ap = a_v[r, pl.ds(c0 + 1, 16)]
            t = csk[s] * a0 + clp[s] * ap + crm[s] * am
            acc = acc + logf * t
        return acc

    def group_body(g_idx, accs):
        r0 = g_idx * _UNROLL
        return tuple(row_terms(r0 + u, accs[u]) for u in range(_UNROLL))

    chunk = min(_CHUNK_ROWS, rows_per_tile)
    accs = (zeros,) * _UNROLL
    for ci in range(rows_per_tile // chunk):
        rows = pl.ds(base + ci * chunk, chunk)
        pltpu.sync_copy(a_hbm.at[rows], a_v)
        pltpu.sync_copy(f_hbm.at[rows], f_v)
        accs = lax.fori_loop(0, chunk // _UNROLL, group_body, accs)
    acc = (accs[0] + accs[1]) + (accs[2] + accs[3])
    accv[...] = acc
    pltpu.sync_copy(accv, out_hbm.at[wid])


def _sc_loss_partials(anchor, feature, nrows):
    atoms = anchor.shape[1]
    rows_per_tile = nrows // _NTILES
    coef = jnp.asarray(_coeff_table())
    mesh = plsc.VectorSubcoreMesh(core_axis_name="c", subcore_axis_name="s")
    run = pl.kernel(
        functools.partial(_sc_body, rows_per_tile),
        out_type=jax.ShapeDtypeStruct((_NTILES, 16), jnp.float32),
        mesh=mesh,
        scratch_types=[
            pltpu.VMEM((min(_CHUNK_ROWS, rows_per_tile), atoms), jnp.float32),
            pltpu.VMEM((min(_CHUNK_ROWS, rows_per_tile), atoms), jnp.float32),
            pltpu.VMEM((12, 16), jnp.float32),
            pltpu.VMEM((16,), jnp.float32),
        ],
    )
    return run(anchor, feature, coef)


def _proj_matrix():
    cs, cl, cr = _col_coeffs()
    w = np.zeros((_ATOMS, _ATOMS), dtype=np.float32)
    for j in range(_ATOMS):
        w[j, j] += cs[j]
        if j > 0:
            w[j, j - 1] += cl[j]
        if j < _ATOMS - 1:
            w[j, j + 1] += cr[j]
    return w


def _tc_body(anchor_ref, feature_ref, w_ref, out_ref):
    i = pl.program_id(0)
    logf = jnp.log(feature_ref[...] + 1e-16)
    proj = lax.dot_general(
        anchor_ref[...], w_ref[...],
        dimension_numbers=(((1,), (0,)), ((), ())),
        preferred_element_type=jnp.float32,
    )
    partial = jnp.sum(proj * logf, keepdims=True)

    @pl.when(i == 0)
    def _init():
        out_ref[...] = jnp.zeros_like(out_ref)

    out_ref[...] += partial


def _tc_loss_sum(anchor, feature, row0, nrows, bm):
    batch, atoms = anchor.shape
    w = jnp.asarray(_proj_matrix())
    blk0 = row0 // bm
    total = pl.pallas_call(
        _tc_body,
        grid=(nrows // bm,),
        in_specs=[
            pl.BlockSpec((bm, atoms), lambda i: (i + blk0, 0)),
            pl.BlockSpec((bm, atoms), lambda i: (i + blk0, 0)),
            pl.BlockSpec((atoms, atoms), lambda i: (0, 0)),
        ],
        out_specs=pl.BlockSpec((1, 1), lambda i: (0, 0)),
        out_shape=jax.ShapeDtypeStruct((1, 1), jnp.float32),
    )(anchor, feature, w)
    return total[0, 0]


_SC_ROWS = 16384


def kernel(anchor, feature):
    batch, _ = anchor.shape
    sc_partials = _sc_loss_partials(anchor, feature, _SC_ROWS)
    if batch > _SC_ROWS:
        tc_total = _tc_loss_sum(anchor, feature, _SC_ROWS, batch - _SC_ROWS, 2048)
    else:
        tc_total = jnp.float32(0.0)
    total = tc_total + jnp.sum(sc_partials)
    return (-total / batch).astype(jnp.float32)


# speedup vs baseline: 2.1638x; 1.3528x over previous
import functools

import jax
import jax.numpy as jnp
import numpy as np
from jax import lax
from jax.experimental import pallas as pl
from jax.experimental.pallas import tpu as pltpu
from jax.experimental.pallas import tpu_sc as plsc

_ATOMS = 51
_V_MAX = 10.0
_V_MIN = -10.0
_SKEW = 0.0

_NTILES = 32
_SLICE_STARTS = (0, 16, 32, 35)

_LOG2_POLY = (
    -0.02456854, 0.2650243, -1.229291, 3.2128003,
    -5.2616825, 6.0668716, -3.0291514,
)
_LN2 = 0.6931471805599453


def _col_coeffs():
    atoms = _ATOMS
    delta = np.float32((_V_MAX - _V_MIN) / (atoms - 1))
    supports = np.linspace(_V_MIN, _V_MAX, atoms).astype(np.float32)
    tz = np.clip(np.float32(_SKEW) + supports, _V_MIN, _V_MAX).astype(np.float32)
    b = ((tz - np.float32(_V_MIN)) / delta).astype(np.float32)
    l = np.floor(b)
    u = np.ceil(b)
    l = np.where((u > 0) & (l == u), l - 1.0, l).astype(np.float32)
    u = np.where((l < atoms - 1) & (l == u), u + 1.0, u).astype(np.float32)
    wl = (u - b).astype(np.float32)
    wu = (b - l).astype(np.float32)
    j = np.arange(atoms, dtype=np.float32)
    l_is_j = l == j
    cs = np.where(l_is_j, wl, wu).astype(np.float32)
    cl = np.where(l_is_j, 0.0, wl).astype(np.float32)
    cr = np.where(l_is_j, wu, 0.0).astype(np.float32)
    return cs, cl, cr


def _coeff_table():
    cs, cl, cr = _col_coeffs()
    atoms = _ATOMS
    csk = cs.copy()
    clp = np.concatenate([cl[1:], [np.float32(0.0)]]).astype(np.float32)
    crm = np.concatenate([[np.float32(0.0)], cr[:-1]]).astype(np.float32)
    tab = np.zeros((12, 16), dtype=np.float32)
    for s, c0 in enumerate(_SLICE_STARTS):
        cols = np.arange(c0, c0 + 16)
        keep = (cols < atoms) if s < 3 else (cols >= 48)
        for g, arr in enumerate((csk, clp, crm)):
            vals = np.where(keep & (cols < atoms), arr[np.minimum(cols, atoms - 1)], 0.0)
            tab[4 * g + s] = vals.astype(np.float32)
    return tab


def _log16(x):
    xi = lax.bitcast_convert_type(x, jnp.int32)
    e = ((xi >> 23) & 0xFF) - 127
    m = lax.bitcast_convert_type((xi & 0x007FFFFF) | 0x3F800000, jnp.float32)
    acc = jnp.full((16,), _LOG2_POLY[0], jnp.float32)
    for c in _LOG2_POLY[1:]:
        acc = acc * m + jnp.float32(c)
    return (e.astype(jnp.float32) + acc) * jnp.float32(_LN2)


_CHUNK_ROWS = 256


_UNROLL = 4


def _gather16(v, idx):
    return lax.gather(
        v, idx.reshape(16, 1),
        dimension_numbers=lax.GatherDimensionNumbers(
            offset_dims=(), collapsed_slice_dims=(0,), start_index_map=(0,)),
        slice_sizes=(1,),
        mode=lax.GatherScatterMode.PROMISE_IN_BOUNDS,
    )


def _sc_body(rows_per_tile, a_hbm, f_hbm, coef_hbm, out_hbm,
             a_v, f_v, coef_v, accv):
    nc = 2
    wid = lax.axis_index("s") * nc + lax.axis_index("c")
    base = wid * rows_per_tile
    pltpu.sync_copy(coef_hbm, coef_v)

    zeros = jnp.zeros((16,), jnp.float32)
    csk = [coef_v[s] for s in range(4)]
    clp = [coef_v[4 + s] for s in range(4)]
    crm = [coef_v[8 + s] for s in range(4)]
    lanes = lax.iota(jnp.int32, 16)
    idx_m = jnp.maximum(lanes - 1, 0)
    idx_p = jnp.minimum(lanes + 1, 15)

    def row_terms(r, acc):
        for s, c0 in enumerate(_SLICE_STARTS):
            logf = _log16(f_v[r, pl.ds(c0, 16)] + jnp.float32(1e-16))
            a0 = a_v[r, pl.ds(c0, 16)]
            if s == 0:
                am = _gather16(a0, idx_m)
            else:
                am = a_v[r, pl.ds(c0 - 1, 16)]
            if s == 3:
                ap = _gather16(a0, idx_p)
            else:
                ap = a_v[r, pl.ds(c0 + 1, 16)]
            t = csk[s] * a0 + clp[s] * ap + crm[s] * am
            acc = acc + logf * t
        return acc

    def group_body(g_idx, accs):
        r0 = g_idx * _UNROLL
        return tuple(row_terms(r0 + u, accs[u]) for u in range(_UNROLL))

    chunk = min(_CHUNK_ROWS, rows_per_tile)
    accs = (zeros,) * _UNROLL
    for ci in range(rows_per_tile // chunk):
        rows = pl.ds(base + ci * chunk, chunk)
        pltpu.sync_copy(a_hbm.at[rows], a_v)
        pltpu.sync_copy(f_hbm.at[rows], f_v)
        accs = lax.fori_loop(0, chunk // _UNROLL, group_body, accs)
    acc = (accs[0] + accs[1]) + (accs[2] + accs[3])
    accv[...] = acc
    pltpu.sync_copy(accv, out_hbm.at[wid])


def _sc_loss_partials(anchor, feature, nrows):
    atoms = anchor.shape[1]
    rows_per_tile = nrows // _NTILES
    coef = jnp.asarray(_coeff_table())
    mesh = plsc.VectorSubcoreMesh(core_axis_name="c", subcore_axis_name="s")
    run = pl.kernel(
        functools.partial(_sc_body, rows_per_tile),
        out_type=jax.ShapeDtypeStruct((_NTILES, 16), jnp.float32),
        mesh=mesh,
        scratch_types=[
            pltpu.VMEM((min(_CHUNK_ROWS, rows_per_tile), atoms), jnp.float32),
            pltpu.VMEM((min(_CHUNK_ROWS, rows_per_tile), atoms), jnp.float32),
            pltpu.VMEM((12, 16), jnp.float32),
            pltpu.VMEM((16,), jnp.float32),
        ],
    )
    return run(anchor, feature, coef)


def _proj_matrix():
    cs, cl, cr = _col_coeffs()
    w = np.zeros((_ATOMS, _ATOMS), dtype=np.float32)
    for j in range(_ATOMS):
        w[j, j] += cs[j]
        if j > 0:
            w[j, j - 1] += cl[j]
        if j < _ATOMS - 1:
            w[j, j + 1] += cr[j]
    return w


def _tc_body(anchor_ref, feature_ref, w_ref, out_ref):
    i = pl.program_id(0)
    logf = jnp.log(feature_ref[...] + 1e-16)
    proj = lax.dot_general(
        anchor_ref[...], w_ref[...],
        dimension_numbers=(((1,), (0,)), ((), ())),
        preferred_element_type=jnp.float32,
    )
    partial = jnp.sum(proj * logf, keepdims=True)

    @pl.when(i == 0)
    def _init():
        out_ref[...] = jnp.zeros_like(out_ref)

    out_ref[...] += partial


def _tc_loss_sum(anchor, feature, row0, nrows, bm):
    batch, atoms = anchor.shape
    w = jnp.asarray(_proj_matrix())
    blk0 = row0 // bm
    total = pl.pallas_call(
        _tc_body,
        grid=(nrows // bm,),
        in_specs=[
            pl.BlockSpec((bm, atoms), lambda i: (i + blk0, 0)),
            pl.BlockSpec((bm, atoms), lambda i: (i + blk0, 0)),
            pl.BlockSpec((atoms, atoms), lambda i: (0, 0)),
        ],
        out_specs=pl.BlockSpec((1, 1), lambda i: (0, 0)),
        out_shape=jax.ShapeDtypeStruct((1, 1), jnp.float32),
    )(anchor, feature, w)
    return total[0, 0]


_SC_ROWS = 2048


def kernel(anchor, feature):
    batch, _ = anchor.shape
    sc_partials = _sc_loss_partials(anchor, feature, _SC_ROWS)
    if batch > _SC_ROWS:
        tc_total = _tc_loss_sum(anchor, feature, _SC_ROWS, batch - _SC_ROWS, 2048)
    else:
        tc_total = jnp.float32(0.0)
    total = tc_total + jnp.sum(sc_partials)
    return (-total / batch).astype(jnp.float32)
